# baseline (device time: 21347 ns/iter reference)
import jax
import jax.numpy as jnp
from jax import lax
from jax.experimental import pallas as pl
from jax.experimental.pallas import tpu as pltpu

N_DEV = 16
EPS = 1e-5
C = 4


def kernel(x, gamma, beta):
    m, n_loc = x.shape
    n_glob = n_loc * N_DEV
    rows_c = m // C
    blk_c = rows_c // 128

    def body(x_hbm, g_ref, b_ref, out_hbm, xv, stats_buf,
             in_sems, out_sems, send_sems, recv_sems):
        me = lax.axis_index("i")

        in_dmas = []
        for k in range(C):
            cp = pltpu.make_async_copy(
                x_hbm.at[pl.ds(k * rows_c, rows_c)],
                xv.at[pl.ds(k * rows_c, rows_c)],
                in_sems.at[k],
            )
            cp.start()
            in_dmas.append(cp)

        barrier = pltpu.get_barrier_semaphore()
        for d in range(1, N_DEV):
            tgt = lax.rem(me + d, N_DEV)
            pl.semaphore_signal(
                barrier, inc=1,
                device_id=(tgt,), device_id_type=pl.DeviceIdType.MESH,
            )

        sends = []
        for k in range(C):
            in_dmas[k].wait()
            xc = xv[k * rows_c:(k + 1) * rows_c, :].reshape(blk_c, 128, n_loc)
            s1 = jnp.sum(xc, axis=2)
            s2 = jnp.sum(xc * xc, axis=2)
            stats_buf[me, k] = jnp.concatenate([s1, s2], axis=0)
            if k == 0:
                pl.semaphore_wait(barrier, N_DEV - 1)
            for d in range(1, N_DEV):
                tgt = lax.rem(me + d, N_DEV)
                r = pltpu.make_async_remote_copy(
                    src_ref=stats_buf.at[me, k],
                    dst_ref=stats_buf.at[me, k],
                    send_sem=send_sems.at[k, d],
                    recv_sem=recv_sems.at[k, me],
                    device_id=(tgt,),
                    device_id_type=pl.DeviceIdType.MESH,
                )
                r.start()
                sends.append(r)

        g = g_ref[:].reshape(1, 1, n_loc)
        b = b_ref[:].reshape(1, 1, n_loc)

        out_dmas = []
        for k in range(C):
            for d in range(1, N_DEV):
                src = lax.rem(me + d, N_DEV)
                recv = pltpu.make_async_remote_copy(
                    src_ref=stats_buf.at[src, k],
                    dst_ref=stats_buf.at[src, k],
                    send_sem=send_sems.at[k, d],
                    recv_sem=recv_sems.at[k, src],
                    device_id=(me,),
                    device_id_type=pl.DeviceIdType.MESH,
                )
                recv.wait_recv()
            tot = jnp.sum(stats_buf[:, k], axis=0)
            mean = (tot[:blk_c, :] / n_glob).reshape(blk_c, 128, 1)
            var = (tot[blk_c:, :] / n_glob).reshape(blk_c, 128, 1) - mean * mean
            rstd = lax.rsqrt(var + EPS)
            xc = xv[k * rows_c:(k + 1) * rows_c, :].reshape(blk_c, 128, n_loc)
            xv[k * rows_c:(k + 1) * rows_c, :] = (
                (xc - mean) * rstd * g + b
            ).reshape(rows_c, n_loc)
            cp = pltpu.make_async_copy(
                xv.at[pl.ds(k * rows_c, rows_c)],
                out_hbm.at[pl.ds(k * rows_c, rows_c)],
                out_sems.at[k],
            )
            cp.start()
            out_dmas.append(cp)

        for r in sends:
            r.wait_send()
        for cp in out_dmas:
            cp.wait()

    return pl.pallas_call(
        body,
        out_shape=jax.ShapeDtypeStruct((m, n_loc), jnp.float32),
        in_specs=[
            pl.BlockSpec(memory_space=pl.ANY),
            pl.BlockSpec(memory_space=pltpu.VMEM),
            pl.BlockSpec(memory_space=pltpu.VMEM),
        ],
        out_specs=pl.BlockSpec(memory_space=pl.ANY),
        scratch_shapes=[
            pltpu.VMEM((m, n_loc), jnp.float32),
            pltpu.VMEM((N_DEV, C, 2 * blk_c, 128), jnp.float32),
            pltpu.SemaphoreType.DMA((C,)),
            pltpu.SemaphoreType.DMA((C,)),
            pltpu.SemaphoreType.DMA((C, N_DEV)),
            pltpu.SemaphoreType.DMA((C, N_DEV)),
        ],
        compiler_params=pltpu.CompilerParams(collective_id=0),
    )(x, gamma.reshape(1, n_loc), beta.reshape(1, n_loc))
